# Initial kernel scaffold; baseline (speedup 1.0000x reference)
import jax
import jax.numpy as jnp
from jax.experimental import pallas as pl


def kernel(embedding, bottom_to_top_paths):
    raise NotImplementedError("write your pallas kernel here")


# speedup vs baseline: 1.8911x; 1.8911x over previous
import functools

import jax
import jax.numpy as jnp
from jax.experimental import pallas as pl


def _layer_body(a_ref, emb_ref, emb_rows_ref, out_ref):
    a = a_ref[...]
    acc = jnp.dot(a, emb_ref[...],
                  preferred_element_type=jnp.float32)
    rowsum = jnp.sum(a, axis=1, keepdims=True)
    out_ref[...] = (emb_rows_ref[...] + acc) * (1.0 / (rowsum + 1.0))


@functools.partial(jax.jit, static_argnames=("bm",))
def _layer(A, emb, bm):
    N, D = emb.shape
    return pl.pallas_call(
        _layer_body,
        grid=(N // bm,),
        in_specs=[
            pl.BlockSpec((bm, N), lambda i: (i, 0)),
            pl.BlockSpec((N, D), lambda i: (0, 0)),
            pl.BlockSpec((bm, D), lambda i: (i, 0)),
        ],
        out_specs=pl.BlockSpec((bm, D), lambda i: (i, 0)),
        out_shape=jax.ShapeDtypeStruct((N, D), jnp.float32),
    )(A, emb, emb)


def kernel(embedding, bottom_to_top_paths):
    emb = embedding
    P = bottom_to_top_paths.shape[0]
    N = emb.shape[0]
    bm = 400 if N % 400 == 0 else 8
    for i in range(P):
        emb = _layer(bottom_to_top_paths[i], emb, bm)
    return emb
